# baseline (speedup 1.0000x reference)
import functools

import jax
import jax.numpy as jnp
from jax import lax
from jax.experimental import pallas as pl
from jax.experimental.pallas import tpu as pltpu
from jax.experimental.pallas import tpu_sc as plsc

VOCAB = 100000
EMBED_DIM = 64
BATCH = 4096
HIST = 50

NC = 2
NS = 16
NW = NC * NS
BW = BATCH // NW
NLG = BW // 16
NBUF = 2

_mesh = plsc.VectorSubcoreMesh(core_axis_name="c", subcore_axis_name="s")


@functools.partial(
    pl.kernel,
    mesh=_mesh,
    out_type=jax.ShapeDtypeStruct((HIST, EMBED_DIM, BATCH), jnp.float32),
    scratch_types=[
        pltpu.VMEM((HIST, BW), jnp.int32),
        pltpu.VMEM((NBUF, BW), jnp.int32),
        pltpu.VMEM((NBUF, BW, 128), jnp.float32),
        pltpu.VMEM((NBUF, EMBED_DIM, BW), jnp.float32),
        [pltpu.SemaphoreType.DMA for _ in range(NBUF)],
        [pltpu.SemaphoreType.DMA for _ in range(NBUF)],
    ],
    compiler_params=pltpu.CompilerParams(
        use_tc_tiling_on_sc=True, needs_layout_passes=False
    ),
)
def _gather_kernel(xt_hbm, w_hbm, out_hbm, idx_v, idx2_v, staged_v, tiles_v,
                   gsems, wsems):
    wid = lax.axis_index("s") * NC + lax.axis_index("c")
    b0 = wid * BW
    iota16 = lax.iota(jnp.int32, 16)

    pltpu.sync_copy(xt_hbm.at[:, pl.ds(b0, BW)], idx_v)

    def start_gather(h, r):
        for lg in range(NLG):
            v = idx_v[h, pl.ds(lg * 16, 16)]
            idx2_v[r, pl.ds(lg * 16, 16)] = lax.shift_right_logical(v, 1)
        pltpu.async_copy(w_hbm.at[idx2_v.at[r]], staged_v.at[r], gsems[r])

    def step(h, r):
        pltpu.make_async_copy(
            w_hbm.at[idx2_v.at[r]], staged_v.at[r], gsems[r]
        ).wait()

        @pl.when(h >= NBUF)
        def _():
            pltpu.make_async_copy(
                tiles_v.at[r], out_hbm.at[0, :, pl.ds(b0, BW)], wsems[r]
            ).wait()

        rows = []
        offs = []
        for lg in range(NLG):
            v = idx_v[h, pl.ds(lg * 16, 16)]
            rows.append(iota16 + (lg * 16))
            offs.append((v & 1) * jnp.int32(EMBED_DIM))

        @pl.loop(0, EMBED_DIM // 8)
        def _(eg):
            cbases = [off + eg * 8 for off in offs]
            erow = eg * 8
            for es in range(8):
                for lg in range(NLG):
                    vec = plsc.load_gather(
                        staged_v.at[r], [rows[lg], cbases[lg] + es]
                    )
                    tiles_v[r, erow + es, pl.ds(lg * 16, 16)] = vec

        pltpu.async_copy(
            tiles_v.at[r], out_hbm.at[h, :, pl.ds(b0, BW)], wsems[r]
        )

        @pl.when(h + NBUF < HIST)
        def _():
            start_gather(h + NBUF, r)

    for r in range(NBUF):
        start_gather(r, r)

    @pl.loop(0, HIST, step=NBUF)
    def _(h):
        for r in range(NBUF):
            step(h + r, r)

    for r in range(NBUF):
        pltpu.make_async_copy(
            tiles_v.at[r], out_hbm.at[0, :, pl.ds(b0, BW)], wsems[r]
        ).wait()


def kernel(x, weights):
    xt = x.T.astype(jnp.int32)
    w128 = weights.reshape(VOCAB // 2, 128)
    res = _gather_kernel(xt, w128)
    return jnp.transpose(res, (2, 0, 1))


# speedup vs baseline: 1.6344x; 1.6344x over previous
import functools

import jax
import jax.numpy as jnp
from jax import lax
from jax.experimental import pallas as pl
from jax.experimental.pallas import tpu as pltpu
from jax.experimental.pallas import tpu_sc as plsc

VOCAB = 100000
EMBED_DIM = 64
TOTAL = 4096 * 50

NC = 2
NS = 16
NW = NC * NS
B_PER_W = TOTAL // NW

CHUNK = 400
NCHUNKS = B_PER_W // CHUNK
NBUF = 4
PREF = 2

_mesh = plsc.VectorSubcoreMesh(core_axis_name="c", subcore_axis_name="s")


@functools.partial(
    pl.kernel,
    mesh=_mesh,
    out_type=jax.ShapeDtypeStruct((TOTAL, EMBED_DIM), jnp.float32),
    scratch_types=[
        pltpu.VMEM((NCHUNKS, CHUNK), jnp.int32),
        pltpu.VMEM((NBUF, CHUNK, EMBED_DIM), jnp.float32),
        [pltpu.SemaphoreType.DMA for _ in range(NBUF)],
        [pltpu.SemaphoreType.DMA for _ in range(NBUF)],
    ],
    compiler_params=pltpu.CompilerParams(use_tc_tiling_on_sc=False),
)
def _gather_kernel(idx_hbm, table_hbm, out_hbm, idx_v, rows_v, gsems, wsems):
    wid = lax.axis_index("s") * NC + lax.axis_index("c")
    base = wid * B_PER_W

    pltpu.sync_copy(idx_hbm.at[wid], idx_v)

    def drain_write(j, ci):
        pltpu.make_async_copy(
            rows_v.at[j], out_hbm.at[pl.ds(base + ci * CHUNK, CHUNK)],
            wsems[j],
        ).wait()

    for b in range(PREF):
        pltpu.async_copy(table_hbm.at[idx_v.at[b]], rows_v.at[b], gsems[b])

    @pl.loop(0, NCHUNKS, step=NBUF)
    def _(g):
        for j in range(NBUF):
            ci = g + j
            pltpu.make_async_copy(
                table_hbm.at[idx_v.at[ci]], rows_v.at[j], gsems[j]
            ).wait()
            pltpu.async_copy(
                rows_v.at[j], out_hbm.at[pl.ds(base + ci * CHUNK, CHUNK)],
                wsems[j],
            )
            nxt = ci + PREF

            @pl.when(nxt < NCHUNKS)
            def _():
                j2 = (j + PREF) % NBUF

                @pl.when(ci >= PREF)
                def _():
                    drain_write(j2, ci - PREF)

                pltpu.async_copy(
                    table_hbm.at[idx_v.at[nxt]], rows_v.at[j2], gsems[j2]
                )

    for j in range(NBUF):
        drain_write(j, NCHUNKS - NBUF + j)


def kernel(x, weights):
    idx = x.reshape(NW, NCHUNKS, CHUNK).astype(jnp.int32)
    out = _gather_kernel(idx, weights)
    return out.reshape(x.shape + (EMBED_DIM,))
